# Initial kernel scaffold; baseline (speedup 1.0000x reference)
import jax
import jax.numpy as jnp
from jax.experimental import pallas as pl


def kernel(xyz, features, sa_W0, sa_b0, sa_g0, sa_be0, sa_W1, sa_b1, sa_g1, sa_be1, sa_W2, sa_b2, sa_g2, sa_be2, fp_W0, fp_b0, fp_g0, fp_be0, fp_W1, fp_b1, fp_g1, fp_be1, c1_W, c1_b, bn1_g, bn1_b, c2_W, c2_b, bn2_g, bn2_b, c3_W, c3_b):
    raise NotImplementedError("write your pallas kernel here")


# speedup vs baseline: 15.4446x; 15.4446x over previous
import functools

import jax
import jax.numpy as jnp
from jax import lax
from jax.experimental import pallas as pl
from jax.experimental.pallas import tpu as pltpu
from jax.experimental.pallas import tpu_sc as plsc

B, N = 4, 8192
S, K = 1024, 32
RADIUS = 0.2
DPAD = 256
F32 = jnp.float32


def _fps_body(x_ref, y_ref, z_ref, nx_ref, ny_ref, nz_ref):
    X = x_ref[...]
    Y = y_ref[...]
    Z = z_ref[...]
    lane = lax.broadcasted_iota(jnp.int32, (B, N), 1)

    def body(i, st):
        dists, far, nx, ny, nz = st
        onehot = lane == far
        cx = jnp.sum(jnp.where(onehot, X, 0.0), axis=1, keepdims=True)
        cy = jnp.sum(jnp.where(onehot, Y, 0.0), axis=1, keepdims=True)
        cz = jnp.sum(jnp.where(onehot, Z, 0.0), axis=1, keepdims=True)
        slane = lax.broadcasted_iota(jnp.int32, (B, S), 1)
        nx = jnp.where(slane == i, cx, nx)
        ny = jnp.where(slane == i, cy, ny)
        nz = jnp.where(slane == i, cz, nz)
        d = (X - cx) ** 2 + (Y - cy) ** 2 + (Z - cz) ** 2
        dists = jnp.minimum(dists, d)
        m = jnp.max(dists, axis=1, keepdims=True)
        far = jnp.min(jnp.where(dists == m, lane, N), axis=1, keepdims=True)
        return (dists, far, nx, ny, nz)

    st = (
        jnp.full((B, N), 1e10, F32),
        jnp.zeros((B, 1), jnp.int32),
        jnp.zeros((B, S), F32),
        jnp.zeros((B, S), F32),
        jnp.zeros((B, S), F32),
    )
    _, _, nx, ny, nz = lax.fori_loop(0, S, body, st)
    nx_ref[...] = nx
    ny_ref[...] = ny
    nz_ref[...] = nz


def _run_fps(X, Y, Z):
    return pl.pallas_call(
        _fps_body,
        out_shape=[jax.ShapeDtypeStruct((B, S), F32)] * 3,
    )(X, Y, Z)


_TS = 128


def _ballq_body(x_ref, y_ref, z_ref, new_ref, idx_ref):
    b = pl.program_id(0)
    ns = new_ref[0]
    cx = ns[:, 0:1]
    cy = ns[:, 1:2]
    cz = ns[:, 2:3]
    X = x_ref[0]
    Y = y_ref[0]
    Z = z_ref[0]
    d2 = (cx - X) ** 2 + (cy - Y) ** 2 + (cz - Z) ** 2
    lane = lax.broadcasted_iota(jnp.int32, (_TS, N), 1).astype(F32)
    score = jnp.where(d2 < RADIUS * RADIUS, lane, lane + N)
    kcol = lax.broadcasted_iota(jnp.int32, (_TS, K), 1)
    idxs = jnp.zeros((_TS, K), F32)
    for k in range(K):
        m = jnp.min(score, axis=1, keepdims=True)
        idxs = jnp.where(kcol == k, m, idxs)
        score = jnp.where(score == m, 2.0 * N + 1.0, score)
    first = idxs[:, 0:1]
    res = jnp.where(idxs < N, idxs, first)
    idx_ref[0] = res.astype(jnp.int32) + b * N


def _run_ballq(X, Y, Z, new3):
    X = X.reshape(B, 1, N)
    Y = Y.reshape(B, 1, N)
    Z = Z.reshape(B, 1, N)
    grid = (B, S // _TS)
    return pl.pallas_call(
        _ballq_body,
        grid=grid,
        in_specs=[
            pl.BlockSpec((1, 1, N), lambda b, j: (b, 0, 0)),
            pl.BlockSpec((1, 1, N), lambda b, j: (b, 0, 0)),
            pl.BlockSpec((1, 1, N), lambda b, j: (b, 0, 0)),
            pl.BlockSpec((1, _TS, 3), lambda b, j: (b, j, 0)),
        ],
        out_specs=pl.BlockSpec((1, _TS, K), lambda b, j: (b, j, 0)),
        out_shape=jax.ShapeDtypeStruct((B, S, K), jnp.int32),
    )(X, Y, Z, new3)


def _sc_gather(table, idx):
    BT = idx.shape[0]
    info = plsc.get_sparse_core_info()
    NW = info.num_cores * info.num_subcores
    b_per_w = BT // NW
    CH = 128
    nch = b_per_w // CH
    mesh = plsc.VectorSubcoreMesh(core_axis_name="c", subcore_axis_name="s")

    @functools.partial(
        pl.kernel,
        mesh=mesh,
        out_type=jax.ShapeDtypeStruct((BT, DPAD), F32),
        scratch_types=[
            pltpu.VMEM((CH,), jnp.int32),
            pltpu.VMEM((CH, DPAD), F32),
            pltpu.SemaphoreType.DMA,
        ],
    )
    def k(table_hbm, idx_hbm, out_hbm, idx_v, rows_v, sem):
        wid = lax.axis_index("s") * info.num_cores + lax.axis_index("c")
        base = wid * b_per_w

        def body(c, carry):
            off = base + c * CH
            pltpu.sync_copy(idx_hbm.at[pl.ds(off, CH)], idx_v)
            pltpu.async_copy(table_hbm.at[idx_v], rows_v, sem).wait()
            pltpu.sync_copy(rows_v, out_hbm.at[pl.ds(off, CH)])
            return carry

        lax.fori_loop(0, nch, body, 0)

    return k(table, idx)


def _bn_relu(y, ssum, ssq, g, be, m_count):
    mean = ssum / m_count
    var = ssq / m_count - mean * mean
    a = g / jnp.sqrt(var + 1e-5)
    return jnp.maximum(y * a + (be - mean * a), 0.0)


def _acc_stats(y, ssum_ref, ssq_ref):
    @pl.when(pl.program_id(0) == 0)
    def _():
        ssum_ref[...] = jnp.zeros_like(ssum_ref)
        ssq_ref[...] = jnp.zeros_like(ssq_ref)

    ssum_ref[...] += jnp.sum(y, axis=0, keepdims=True)
    ssq_ref[...] += jnp.sum(y * y, axis=0, keepdims=True)


_STAT = [
    jax.ShapeDtypeStruct((1, 128), F32),
    jax.ShapeDtypeStruct((1, 128), F32),
]
_C = pl.BlockSpec((1, 128), lambda j: (0, 0))
_CS = pl.BlockSpec((1, 128), lambda j: (0, 0))


_TR = 2048


def _sa1_body(g_ref, ns_ref, w_ref, b_ref, y_ref, ssum_ref, ssq_ref):
    w = w_ref[...].astype(jnp.bfloat16)
    g = g_ref[...]
    ns_e = jnp.broadcast_to(
        ns_ref[...][:, None, :], (_TR // K, K, DPAD)
    ).reshape(_TR, DPAD)
    xn = (g - ns_e) / RADIUS
    lane = lax.broadcasted_iota(jnp.int32, (_TR, DPAD), 1)
    xin = jnp.where(lane < 3, xn, g).astype(jnp.bfloat16)
    y = jnp.dot(xin, w, preferred_element_type=F32) + b_ref[...]
    y_ref[...] = y
    _acc_stats(y, ssum_ref, ssq_ref)


def _run_sa1(G, ns_flat, W0s, b0):
    grid = (B * S * K // _TR,)
    return pl.pallas_call(
        _sa1_body,
        grid=grid,
        in_specs=[
            pl.BlockSpec((_TR, DPAD), lambda j: (j, 0)),
            pl.BlockSpec((_TR // K, DPAD), lambda j: (j, 0)),
            pl.BlockSpec((DPAD, 128), lambda j: (0, 0)),
            _C,
        ],
        out_specs=[pl.BlockSpec((_TR, 128), lambda j: (j, 0)), _CS, _CS],
        out_shape=[jax.ShapeDtypeStruct((B * S * K, 128), F32)] + _STAT,
    )(G, ns_flat, W0s, b0)


def _bnmm_body(m_count, emit_z, *refs):
    if emit_z:
        (y_ref, su_ref, sq_ref, g_ref, be_ref, w_ref, b_ref,
         z_ref, yo_ref, nsu_ref, nsq_ref) = refs
    else:
        (y_ref, su_ref, sq_ref, g_ref, be_ref, w_ref, b_ref,
         yo_ref, nsu_ref, nsq_ref) = refs
    z = _bn_relu(y_ref[...], su_ref[...], sq_ref[...], g_ref[...],
                 be_ref[...], m_count)
    if emit_z:
        z_ref[...] = z
    y = jnp.dot(
        z.astype(jnp.bfloat16), w_ref[...].astype(jnp.bfloat16),
        preferred_element_type=F32,
    ) + b_ref[...]
    yo_ref[...] = y
    _acc_stats(y, nsu_ref, nsq_ref)


def _run_bnmm(y, su, sq, g, be, Wt, b, m_count, rows, emit_z=False):
    grid = (rows // _TR,)
    row_spec = pl.BlockSpec((_TR, 128), lambda j: (j, 0))
    out_specs = [row_spec, _CS, _CS]
    out_shape = [jax.ShapeDtypeStruct((rows, 128), F32)] + _STAT
    if emit_z:
        out_specs = [row_spec] + out_specs
        out_shape = [jax.ShapeDtypeStruct((rows, 128), F32)] + out_shape
    return pl.pallas_call(
        functools.partial(_bnmm_body, m_count, emit_z),
        grid=grid,
        in_specs=[
            row_spec, _C, _C, _C, _C,
            pl.BlockSpec((128, 128), lambda j: (0, 0)),
            _C,
        ],
        out_specs=out_specs,
        out_shape=out_shape,
    )(y, su, sq, g, be, Wt, b)


def _sa4_body(m_count, y_ref, su_ref, sq_ref, g_ref, be_ref, o_ref):
    z = _bn_relu(y_ref[...], su_ref[...], sq_ref[...], g_ref[...],
                 be_ref[...], m_count)
    o_ref[...] = jnp.max(z.reshape(_TR // K, K, 128), axis=1)


def _run_sa4(y3, su, sq, g, be):
    grid = (B * S * K // _TR,)
    return pl.pallas_call(
        functools.partial(_sa4_body, float(B * S * K)),
        grid=grid,
        in_specs=[pl.BlockSpec((_TR, 128), lambda j: (j, 0)), _C, _C, _C, _C],
        out_specs=pl.BlockSpec((_TR // K, 128), lambda j: (j, 0)),
        out_shape=jax.ShapeDtypeStruct((B * S, 128), F32),
    )(y3, su, sq, g, be)


_TN = 256


def _interp_body(xyzp_ref, nx_ref, ny_ref, nz_ref, fsa_ref, o_ref):
    p = xyzp_ref[0]
    px = p[:, 0:1]
    py = p[:, 1:2]
    pz = p[:, 2:3]
    d2 = (
        (px - nx_ref[0]) ** 2
        + (py - ny_ref[0]) ** 2
        + (pz - nz_ref[0]) ** 2
    )
    lane = lax.broadcasted_iota(jnp.int32, (_TN, S), 1)
    work = d2
    ds = []
    its = []
    for _ in range(3):
        m = jnp.min(work, axis=1, keepdims=True)
        it = jnp.min(jnp.where(work == m, lane, S), axis=1, keepdims=True)
        ds.append(m)
        its.append(it)
        work = jnp.where(lane == it, jnp.float32(1e30), work)
    recip = [1.0 / (d + 1e-8) for d in ds]
    tot = recip[0] + recip[1] + recip[2]
    Wm = jnp.zeros((_TN, S), F32)
    for r, it in zip(recip, its):
        Wm = Wm + jnp.where(lane == it, r / tot, 0.0)
    o_ref[...] = jnp.dot(Wm, fsa_ref[0], preferred_element_type=F32, precision=jax.lax.Precision.HIGHEST)


def _run_interp(xyz3, nX, nY, nZ, fsa3):
    nX = nX.reshape(B, 1, S)
    nY = nY.reshape(B, 1, S)
    nZ = nZ.reshape(B, 1, S)
    nb = N // _TN
    grid = (B * nb,)
    return pl.pallas_call(
        _interp_body,
        grid=grid,
        in_specs=[
            pl.BlockSpec((1, _TN, 3), lambda j: (j // nb, j % nb, 0)),
            pl.BlockSpec((1, 1, S), lambda j: (j // nb, 0, 0)),
            pl.BlockSpec((1, 1, S), lambda j: (j // nb, 0, 0)),
            pl.BlockSpec((1, 1, S), lambda j: (j // nb, 0, 0)),
            pl.BlockSpec((1, S, 128), lambda j: (j // nb, 0, 0)),
        ],
        out_specs=pl.BlockSpec((_TN, 128), lambda j: (j, 0)),
        out_shape=jax.ShapeDtypeStruct((B * N, 128), F32),
    )(xyz3, nX, nY, nZ, fsa3)


def _fp1_body(i_ref, f_ref, wa_ref, wb_ref, b_ref, y_ref, su_ref, sq_ref):
    bf = jnp.bfloat16
    y = (
        jnp.dot(i_ref[...].astype(bf), wa_ref[...].astype(bf),
                preferred_element_type=F32)
        + jnp.dot(f_ref[...].astype(bf), wb_ref[...].astype(bf),
                  preferred_element_type=F32)
        + b_ref[...]
    )
    y_ref[...] = y
    _acc_stats(y, su_ref, sq_ref)


def _run_fp1(interp, featsT, Wa, Wb, b):
    grid = (B * N // _TR,)
    row_spec = pl.BlockSpec((_TR, 128), lambda j: (j, 0))
    return pl.pallas_call(
        _fp1_body,
        grid=grid,
        in_specs=[
            row_spec, row_spec,
            pl.BlockSpec((128, 128), lambda j: (0, 0)),
            pl.BlockSpec((128, 128), lambda j: (0, 0)),
            _C,
        ],
        out_specs=[row_spec, _CS, _CS],
        out_shape=[jax.ShapeDtypeStruct((B * N, 128), F32)] + _STAT,
    )(interp, featsT, Wa, Wb, b)


def _fp5_body(y_ref, su_ref, sq_ref, g_ref, be_ref, wo_ref, ws_ref, wv_ref,
              bo_ref, bs_ref, bv_ref, seed_ref, xyz_ref,
              vx_ref, off_ref, sc_ref, vf_ref):
    z = _bn_relu(y_ref[...], su_ref[...], sq_ref[...], g_ref[...],
                 be_ref[...], float(B * N))
    zb = z.astype(jnp.bfloat16)
    bf = jnp.bfloat16
    off = jnp.dot(zb, wo_ref[...].astype(bf), preferred_element_type=F32) + bo_ref[...]
    s = jnp.dot(zb, ws_ref[...].astype(bf), preferred_element_type=F32) + bs_ref[...]
    v = jnp.dot(zb, wv_ref[...].astype(bf), preferred_element_type=F32) + bv_ref[...]
    off_ref[...] = off
    vx_ref[...] = xyz_ref[...] + off
    sc_ref[...] = 1.0 / (1.0 + jnp.exp(-s))
    vf = seed_ref[...] + v
    nrm = jnp.sqrt(jnp.sum(vf * vf, axis=1, keepdims=True))
    vf_ref[...] = vf / jnp.maximum(nrm, 1e-12)


def _run_fp5(y4, su, sq, g, be, Wo, Ws, Wv, bo, bs, bv, seed, xyz_flat):
    grid = (B * N // _TR,)
    row_spec = pl.BlockSpec((_TR, 128), lambda j: (j, 0))
    spec3 = pl.BlockSpec((_TR, 3), lambda j: (j, 0))
    spec1 = pl.BlockSpec((_TR, 1), lambda j: (j, 0))
    return pl.pallas_call(
        _fp5_body,
        grid=grid,
        in_specs=[
            row_spec, _C, _C, _C, _C,
            pl.BlockSpec((128, 3), lambda j: (0, 0)),
            pl.BlockSpec((128, 1), lambda j: (0, 0)),
            pl.BlockSpec((128, 128), lambda j: (0, 0)),
            pl.BlockSpec((1, 3), lambda j: (0, 0)),
            pl.BlockSpec((1, 1), lambda j: (0, 0)),
            _C,
            row_spec, spec3,
        ],
        out_specs=[spec3, spec3, spec1, row_spec],
        out_shape=[
            jax.ShapeDtypeStruct((B * N, 3), F32),
            jax.ShapeDtypeStruct((B * N, 3), F32),
            jax.ShapeDtypeStruct((B * N, 1), F32),
            jax.ShapeDtypeStruct((B * N, 128), F32),
        ],
    )(y4, su, sq, g, be, Wo, Ws, Wv, bo, bs, bv, seed, xyz_flat)


def kernel(xyz, features, sa_W0, sa_b0, sa_g0, sa_be0, sa_W1, sa_b1, sa_g1,
           sa_be1, sa_W2, sa_b2, sa_g2, sa_be2, fp_W0, fp_b0, fp_g0, fp_be0,
           fp_W1, fp_b1, fp_g1, fp_be1, c1_W, c1_b, bn1_g, bn1_b, c2_W, c2_b,
           bn2_g, bn2_b, c3_W, c3_b):
    X = xyz[..., 0]
    Y = xyz[..., 1]
    Z = xyz[..., 2]
    featsT = jnp.transpose(features, (0, 2, 1))
    table = jnp.concatenate(
        [xyz, featsT, jnp.zeros((B, N, DPAD - 131), F32)], axis=-1
    ).reshape(B * N, DPAD)

    nX, nY, nZ = _run_fps(X, Y, Z)
    new3 = jnp.stack([nX, nY, nZ], axis=-1)
    idx = _run_ballq(X, Y, Z, new3)
    G = _sc_gather(table, idx.reshape(B * S * K))

    W0s = jnp.concatenate(
        [sa_W0, jnp.zeros((128, DPAD - 131), F32)], axis=1
    ).T
    ns_pad = jnp.pad(new3.reshape(B * S, 3), ((0, 0), (0, DPAD - 3)))
    r = lambda v: v.reshape(1, -1)
    M_SA = float(B * S * K)

    y1, su1, sq1 = _run_sa1(G, ns_pad, W0s, r(sa_b0))
    y2, su2, sq2 = _run_bnmm(y1, su1, sq1, r(sa_g0), r(sa_be0), sa_W1.T,
                             r(sa_b1), M_SA, B * S * K)
    y3, su3, sq3 = _run_bnmm(y2, su2, sq2, r(sa_g1), r(sa_be1), sa_W2.T,
                             r(sa_b2), M_SA, B * S * K)
    feat_sa = _run_sa4(y3, su3, sq3, r(sa_g2), r(sa_be2))

    interp = _run_interp(xyz, nX, nY, nZ, feat_sa.reshape(B, S, 128))

    M_FP = float(B * N)
    yf0, fu0, fq0 = _run_fp1(interp, featsT.reshape(B * N, 128),
                             fp_W0[:, :128].T, fp_W0[:, 128:].T, r(fp_b0))
    yf1, fu1, fq1 = _run_bnmm(yf0, fu0, fq0, r(fp_g0), r(fp_be0), fp_W1.T,
                              r(fp_b1), M_FP, B * N)
    seed, yc1, cu1, cq1 = _run_bnmm(yf1, fu1, fq1, r(fp_g1), r(fp_be1),
                                    c1_W.T, r(c1_b), M_FP, B * N, emit_z=True)
    yc2, cu2, cq2 = _run_bnmm(yc1, cu1, cq1, r(bn1_g), r(bn1_b), c2_W.T,
                              r(c2_b), M_FP, B * N)
    vx, off, sc, vf = _run_fp5(
        yc2, cu2, cq2, r(bn2_g), r(bn2_b),
        c3_W[0:3, :].T, c3_W[3:4, :].T, c3_W[4:, :].T,
        c3_b[0:3].reshape(1, 3), c3_b[3:4].reshape(1, 1), r(c3_b[4:]),
        seed, xyz.reshape(B * N, 3),
    )

    return (
        vx.reshape(B, N, 3),
        off.reshape(B, N, 3),
        sc.reshape(B, N),
        vf.reshape(B, N, 128),
    )
